# baseline (speedup 1.0000x reference)
import functools
import math

import jax
import jax.numpy as jnp
from jax import lax
from jax.experimental import pallas as pl
from jax.experimental.pallas import tpu as pltpu
from jax.experimental.pallas import tpu_sc as plsc

_V = 100000
_B = 1024
_EPS = 0.1
_PAD = 0
_S = _EPS / (_V - 2)
_C1 = (_V - 2) * _S * math.log(_S) + (1.0 - _EPS) * math.log(1.0 - _EPS)

_CH = 8
_NSTEP = _B // _CH
_NBUF = 8

_TCUT = (_V // 128) * 128
_TAIL = _V - _TCUT

_NC = 2
_NS = 16
_NW = _NC * _NS
_RPW = _B // _NW
_DEPTH = 4


def _tc_partial_body(t_ref, x_hbm, o_ref, buf, sem):
    step = pl.program_id(0)

    def copy_chunk(j, slot):
        return pltpu.make_async_copy(
            x_hbm.at[pl.ds(pl.multiple_of(j * _CH, _CH), _CH)],
            buf.at[slot], sem.at[slot])

    @pl.when(step == 0)
    def _():
        o_ref[0, 0, 0] = 0.0
        o_ref[0, 0, 1] = 0.0
        o_ref[0, 0, 2] = 0.0
        for s in range(_NBUF):
            copy_chunk(s, s).start()

    slot = lax.rem(step, _NBUF)
    copy_chunk(step, slot).wait()

    x = buf[slot]
    t = t_ref[0]
    valid = (t != _PAD).astype(jnp.float32)
    rs = jnp.sum(x, axis=1, keepdims=True) - x[:, 0:1]
    colt = jax.lax.broadcasted_iota(jnp.int32, (_CH, _TAIL), 1) + _TCUT
    gtail = jnp.where(colt == t, x[:, _TCUT:], 0.0) * valid
    o_ref[0, 0, 0] += jnp.sum(valid * rs)
    o_ref[0, 0, 1] += jnp.sum(valid)
    o_ref[0, 0, 2] += jnp.sum(gtail)

    nxt = step + _NBUF

    @pl.when(nxt < _NSTEP)
    def _():
        copy_chunk(nxt, slot).start()


def _sc_body(x_hbm, t_hbm, out_hbm, tv, b0, b1, b2, b3, stage,
             s0, s1, s2, s3):
    wid = lax.axis_index("s") * _NC + lax.axis_index("c")
    base = wid * _RPW

    pltpu.sync_copy(t_hbm.at[pl.ds(base, _RPW)], tv)
    lane = lax.iota(jnp.int32, 16)
    tvecs = [tv[pl.ds(0, 16)], tv[pl.ds(16, 16)]]
    ts = [tvecs[i // 16][i % 16] for i in range(_RPW)]

    bufs = (b0, b1, b2, b3)
    sems = (s0, s1, s2, s3)

    def mk(i):
        cstart = pl.multiple_of(
            jnp.minimum(ts[i], _TCUT - 1) & jnp.int32(-128), 128)
        rstart = pl.multiple_of(base + (i // 8) * 8, 8)
        return pltpu.make_async_copy(
            x_hbm.at[pl.ds(rstart, 8), pl.ds(cstart, 128)],
            bufs[i % _DEPTH], sems[i % _DEPTH])

    copies = [None] * _RPW
    for i in range(_DEPTH):
        copies[i] = mk(i)
        copies[i].start()

    acc = jnp.zeros((16,), jnp.float32)
    for i in range(_RPW):
        copies[i].wait()
        valid = (ts[i] != _PAD) & (ts[i] < _TCUT)
        loff = jnp.where(valid, ts[i] & jnp.int32(127), jnp.int32(-1))
        for k in range(8):
            v = bufs[i % _DEPTH][i % 8, pl.ds(k * 16, 16)]
            m = (lane + (k * 16)) == loff
            acc = acc + jnp.where(m, v, 0.0)
        nxt = i + _DEPTH
        if nxt < _RPW:
            copies[nxt] = mk(nxt)
            copies[nxt].start()

    stage[...] = acc
    pltpu.sync_copy(stage, out_hbm.at[wid])


_sc_gather = functools.partial(
    pl.kernel,
    out_type=jax.ShapeDtypeStruct((_NW, 16), jnp.float32),
    mesh=plsc.VectorSubcoreMesh(core_axis_name="c", subcore_axis_name="s"),
    scratch_types=[
        pltpu.VMEM((_RPW,), jnp.int32),
        pltpu.VMEM((8, 128), jnp.float32),
        pltpu.VMEM((8, 128), jnp.float32),
        pltpu.VMEM((8, 128), jnp.float32),
        pltpu.VMEM((8, 128), jnp.float32),
        pltpu.VMEM((16,), jnp.float32),
        pltpu.SemaphoreType.DMA,
        pltpu.SemaphoreType.DMA,
        pltpu.SemaphoreType.DMA,
        pltpu.SemaphoreType.DMA,
    ],
)(_sc_body)


def _combine_body(p_ref, g_ref, o_ref):
    p = p_ref[...]
    g = g_ref[...]
    a = p[0, 0, 0]
    n = p[0, 0, 1]
    gsum = jnp.sum(g) + p[0, 0, 2]
    o_ref[0, 0] = _C1 - (_S * a + (1.0 - _EPS - _S) * gsum) / n


def kernel(output, target):
    target = target.astype(jnp.int32)
    t3 = target.reshape(_NSTEP, _CH, 1)
    tc_partials = pl.pallas_call(
        _tc_partial_body,
        grid=(_NSTEP,),
        in_specs=[
            pl.BlockSpec((1, _CH, 1), lambda i: (i, 0, 0)),
            pl.BlockSpec(memory_space=pl.ANY),
        ],
        out_specs=pl.BlockSpec((1, 1, 3), lambda i: (0, 0, 0),
                               memory_space=pltpu.SMEM),
        out_shape=jax.ShapeDtypeStruct((1, 1, 3), jnp.float32),
        scratch_shapes=[
            pltpu.VMEM((_NBUF, _CH, _V), jnp.float32),
            pltpu.SemaphoreType.DMA((_NBUF,)),
        ],
        compiler_params=pltpu.CompilerParams(
            dimension_semantics=("arbitrary",),
        ),
    )(t3, output)
    sc_partials = _sc_gather(output, target)
    res = pl.pallas_call(
        _combine_body,
        out_specs=pl.BlockSpec(memory_space=pltpu.SMEM),
        out_shape=jax.ShapeDtypeStruct((1, 1), jnp.float32),
    )(tc_partials, sc_partials)
    return res[0, 0]


# speedup vs baseline: 1.0870x; 1.0870x over previous
import functools
import math

import jax
import jax.numpy as jnp
from jax import lax
from jax.experimental import pallas as pl
from jax.experimental.pallas import tpu as pltpu
from jax.experimental.pallas import tpu_sc as plsc

_V = 100000
_B = 1024
_EPS = 0.1
_PAD = 0
_S = _EPS / (_V - 2)
_C1 = (_V - 2) * _S * math.log(_S) + (1.0 - _EPS) * math.log(1.0 - _EPS)

_RBLK = 32
_NBLK = _B // _RBLK

_TCUT = (_V // 128) * 128
_TAIL = _V - _TCUT

_NC = 2
_NS = 16
_NW = _NC * _NS
_RPW = _B // _NW
_DEPTH = 4


def _tc_partial_body(t_ref, x_ref, o_ref):
    x = x_ref[...]
    t = t_ref[0]
    valid = (t != _PAD).astype(jnp.float32)
    rs = jnp.sum(x, axis=1, keepdims=True) - x[:, 0:1]
    colt = jax.lax.broadcasted_iota(jnp.int32, (_RBLK, _TAIL), 1) + _TCUT
    gtail = jnp.where(colt == t, x[:, _TCUT:], 0.0) * valid
    o_ref[0, 0, 0] = jnp.sum(valid * rs)
    o_ref[0, 0, 1] = jnp.sum(valid)
    o_ref[0, 0, 2] = jnp.sum(gtail)


def _sc_body(x_hbm, t_hbm, out_hbm, tv, b0, b1, b2, b3, stage,
             s0, s1, s2, s3):
    wid = lax.axis_index("s") * _NC + lax.axis_index("c")
    base = wid * _RPW

    pltpu.sync_copy(t_hbm.at[pl.ds(base, _RPW)], tv)
    lane = lax.iota(jnp.int32, 16)
    tvecs = [tv[pl.ds(0, 16)], tv[pl.ds(16, 16)]]
    ts = [tvecs[i // 16][i % 16] for i in range(_RPW)]

    bufs = (b0, b1, b2, b3)
    sems = (s0, s1, s2, s3)

    def mk(i):
        cstart = pl.multiple_of(
            jnp.minimum(ts[i], _TCUT - 1) & jnp.int32(-128), 128)
        rstart = pl.multiple_of(base + (i // 8) * 8, 8)
        return pltpu.make_async_copy(
            x_hbm.at[pl.ds(rstart, 8), pl.ds(cstart, 128)],
            bufs[i % _DEPTH], sems[i % _DEPTH])

    copies = [None] * _RPW
    for i in range(_DEPTH):
        copies[i] = mk(i)
        copies[i].start()

    acc = jnp.zeros((16,), jnp.float32)
    for i in range(_RPW):
        copies[i].wait()
        valid = (ts[i] != _PAD) & (ts[i] < _TCUT)
        loff = jnp.where(valid, ts[i] & jnp.int32(127), jnp.int32(-1))
        for k in range(8):
            v = bufs[i % _DEPTH][i % 8, pl.ds(k * 16, 16)]
            m = (lane + (k * 16)) == loff
            acc = acc + jnp.where(m, v, 0.0)
        nxt = i + _DEPTH
        if nxt < _RPW:
            copies[nxt] = mk(nxt)
            copies[nxt].start()

    stage[...] = acc
    pltpu.sync_copy(stage, out_hbm.at[wid])


_sc_gather = functools.partial(
    pl.kernel,
    out_type=jax.ShapeDtypeStruct((_NW, 16), jnp.float32),
    mesh=plsc.VectorSubcoreMesh(core_axis_name="c", subcore_axis_name="s"),
    scratch_types=[
        pltpu.VMEM((_RPW,), jnp.int32),
        pltpu.VMEM((8, 128), jnp.float32),
        pltpu.VMEM((8, 128), jnp.float32),
        pltpu.VMEM((8, 128), jnp.float32),
        pltpu.VMEM((8, 128), jnp.float32),
        pltpu.VMEM((16,), jnp.float32),
        pltpu.SemaphoreType.DMA,
        pltpu.SemaphoreType.DMA,
        pltpu.SemaphoreType.DMA,
        pltpu.SemaphoreType.DMA,
    ],
)(_sc_body)


def _combine_body(p_ref, g_ref, o_ref):
    p = p_ref[...]
    g = g_ref[...]
    a = jnp.sum(p[:, 0, 0])
    n = jnp.sum(p[:, 0, 1])
    gsum = jnp.sum(g) + jnp.sum(p[:, 0, 2])
    o_ref[0, 0] = _C1 - (_S * a + (1.0 - _EPS - _S) * gsum) / n


def kernel(output, target):
    target = target.astype(jnp.int32)
    t3 = target.reshape(_NBLK, _RBLK, 1)
    tc_partials = pl.pallas_call(
        _tc_partial_body,
        grid=(_NBLK,),
        in_specs=[
            pl.BlockSpec((1, _RBLK, 1), lambda i: (i, 0, 0)),
            pl.BlockSpec((_RBLK, _V), lambda i: (i, 0)),
        ],
        out_specs=pl.BlockSpec((1, 1, 3), lambda i: (i, 0, 0),
                               memory_space=pltpu.SMEM),
        out_shape=jax.ShapeDtypeStruct((_NBLK, 1, 3), jnp.float32),
        compiler_params=pltpu.CompilerParams(
            dimension_semantics=("parallel",),
        ),
    )(t3, output)
    sc_partials = _sc_gather(output, target)
    res = pl.pallas_call(
        _combine_body,
        out_specs=pl.BlockSpec(memory_space=pltpu.SMEM),
        out_shape=jax.ShapeDtypeStruct((1, 1), jnp.float32),
    )(tc_partials, sc_partials)
    return res[0, 0]


# speedup vs baseline: 1.0871x; 1.0001x over previous
import functools
import math

import jax
import jax.numpy as jnp
from jax import lax
from jax.experimental import pallas as pl
from jax.experimental.pallas import tpu as pltpu
from jax.experimental.pallas import tpu_sc as plsc

_V = 100000
_B = 1024
_EPS = 0.1
_PAD = 0
_S = _EPS / (_V - 2)
_C1 = (_V - 2) * _S * math.log(_S) + (1.0 - _EPS) * math.log(1.0 - _EPS)

_RBLK = 64
_NBLK = _B // _RBLK
_NSTRM = 4
_SRBLK = _RBLK // _NSTRM

_TCUT = (_V // 128) * 128
_TAIL = _V - _TCUT

_NC = 2
_NS = 16
_NW = _NC * _NS
_RPW = _B // _NW
_DEPTH = 4


def _tc_partial_body(t_ref, *refs):
    xs = refs[:_NSTRM]
    o_ref = refs[_NSTRM]
    t = t_ref[0]
    colt = jax.lax.broadcasted_iota(jnp.int32, (_SRBLK, _TAIL), 1) + _TCUT
    a = jnp.float32(0.0)
    g = jnp.float32(0.0)
    for k in range(_NSTRM):
        x = xs[k][...]
        tk = t[k * _SRBLK:(k + 1) * _SRBLK]
        valid = (tk != _PAD).astype(jnp.float32)
        rs = jnp.sum(x, axis=1, keepdims=True) - x[:, 0:1]
        gtail = jnp.where(colt == tk, x[:, _TCUT:], 0.0) * valid
        a += jnp.sum(valid * rs)
        g += jnp.sum(gtail)
    o_ref[0, 0, 0] = a
    o_ref[0, 0, 1] = jnp.sum((t != _PAD).astype(jnp.float32))
    o_ref[0, 0, 2] = g


def _sc_body(x_hbm, t_hbm, out_hbm, tv, b0, b1, b2, b3, stage,
             s0, s1, s2, s3):
    wid = lax.axis_index("s") * _NC + lax.axis_index("c")
    base = wid * _RPW

    pltpu.sync_copy(t_hbm.at[pl.ds(base, _RPW)], tv)
    lane = lax.iota(jnp.int32, 16)
    tvecs = [tv[pl.ds(0, 16)], tv[pl.ds(16, 16)]]
    ts = [tvecs[i // 16][i % 16] for i in range(_RPW)]

    bufs = (b0, b1, b2, b3)
    sems = (s0, s1, s2, s3)

    def mk(i):
        cstart = pl.multiple_of(
            jnp.minimum(ts[i], _TCUT - 1) & jnp.int32(-128), 128)
        rstart = pl.multiple_of(base + (i // 8) * 8, 8)
        return pltpu.make_async_copy(
            x_hbm.at[pl.ds(rstart, 8), pl.ds(cstart, 128)],
            bufs[i % _DEPTH], sems[i % _DEPTH])

    copies = [None] * _RPW
    for i in range(_DEPTH):
        copies[i] = mk(i)
        copies[i].start()

    acc = jnp.zeros((16,), jnp.float32)
    for i in range(_RPW):
        copies[i].wait()
        valid = (ts[i] != _PAD) & (ts[i] < _TCUT)
        loff = jnp.where(valid, ts[i] & jnp.int32(127), jnp.int32(-1))
        for k in range(8):
            v = bufs[i % _DEPTH][i % 8, pl.ds(k * 16, 16)]
            m = (lane + (k * 16)) == loff
            acc = acc + jnp.where(m, v, 0.0)
        nxt = i + _DEPTH
        if nxt < _RPW:
            copies[nxt] = mk(nxt)
            copies[nxt].start()

    stage[...] = acc
    pltpu.sync_copy(stage, out_hbm.at[wid])


_sc_gather = functools.partial(
    pl.kernel,
    out_type=jax.ShapeDtypeStruct((_NW, 16), jnp.float32),
    mesh=plsc.VectorSubcoreMesh(core_axis_name="c", subcore_axis_name="s"),
    scratch_types=[
        pltpu.VMEM((_RPW,), jnp.int32),
        pltpu.VMEM((8, 128), jnp.float32),
        pltpu.VMEM((8, 128), jnp.float32),
        pltpu.VMEM((8, 128), jnp.float32),
        pltpu.VMEM((8, 128), jnp.float32),
        pltpu.VMEM((16,), jnp.float32),
        pltpu.SemaphoreType.DMA,
        pltpu.SemaphoreType.DMA,
        pltpu.SemaphoreType.DMA,
        pltpu.SemaphoreType.DMA,
    ],
)(_sc_body)


def _combine_body(p_ref, g_ref, o_ref):
    p = p_ref[...]
    g = g_ref[...]
    a = jnp.sum(p[:, 0, 0])
    n = jnp.sum(p[:, 0, 1])
    gsum = jnp.sum(g) + jnp.sum(p[:, 0, 2])
    o_ref[0, 0] = _C1 - (_S * a + (1.0 - _EPS - _S) * gsum) / n


def kernel(output, target):
    target = target.astype(jnp.int32)
    t3 = target.reshape(_NBLK, _RBLK, 1)
    tc_partials = pl.pallas_call(
        _tc_partial_body,
        grid=(_NBLK,),
        in_specs=[pl.BlockSpec((1, _RBLK, 1), lambda i: (i, 0, 0))] + [
            pl.BlockSpec((_SRBLK, _V), lambda i, k=k: (i * _NSTRM + k, 0))
            for k in range(_NSTRM)
        ],
        out_specs=pl.BlockSpec((1, 1, 3), lambda i: (i, 0, 0),
                               memory_space=pltpu.SMEM),
        out_shape=jax.ShapeDtypeStruct((_NBLK, 1, 3), jnp.float32),
        compiler_params=pltpu.CompilerParams(
            dimension_semantics=("parallel",),
        ),
    )(t3, *([output] * _NSTRM))
    sc_partials = _sc_gather(output, target)
    res = pl.pallas_call(
        _combine_body,
        out_specs=pl.BlockSpec(memory_space=pltpu.SMEM),
        out_shape=jax.ShapeDtypeStruct((1, 1), jnp.float32),
    )(tc_partials, sc_partials)
    return res[0, 0]


# speedup vs baseline: 1.1155x; 1.0261x over previous
import math

import jax
import jax.numpy as jnp
from jax.experimental import pallas as pl
from jax.experimental.pallas import tpu as pltpu

_V = 100000
_B = 1024
_EPS = 0.1
_PAD = 0
_S = _EPS / (_V - 2)
_C1 = (_V - 2) * _S * math.log(_S) + (1.0 - _EPS) * math.log(1.0 - _EPS)

_RBLK = 64
_NBLK = _B // _RBLK


_NSTRM = 4
_SRBLK = _RBLK // _NSTRM


def _partial_body(t_ref, *refs):
    xs = refs[:_NSTRM]
    o_ref = refs[_NSTRM]
    t = t_ref[0]
    col = jax.lax.broadcasted_iota(jnp.int32, (_SRBLK, _V), 1)
    acc = jnp.float32(0.0)
    for k in range(_NSTRM):
        tk = t[k * _SRBLK:(k + 1) * _SRBLK]
        w = jnp.where(col == tk, 1.0 - _EPS,
                      jnp.where(col == _PAD, 0.0, _S))
        w = jnp.where(tk == _PAD, 0.0, w)
        acc += jnp.sum(w * xs[k][...])
    o_ref[0, 0, 0] = acc
    o_ref[0, 0, 1] = jnp.sum((t != _PAD).astype(jnp.float32))


def _combine_body(p_ref, o_ref):
    p = p_ref[...]
    o_ref[0, 0] = _C1 - jnp.sum(p[:, 0, 0]) / jnp.sum(p[:, 0, 1])


def kernel(output, target):
    t3 = target.astype(jnp.int32).reshape(_NBLK, _RBLK, 1)
    partials = pl.pallas_call(
        _partial_body,
        grid=(_NBLK,),
        in_specs=[pl.BlockSpec((1, _RBLK, 1), lambda i: (i, 0, 0))] + [
            pl.BlockSpec((_SRBLK, _V),
                         lambda i, k=k: (i * _NSTRM + k, 0))
            for k in range(_NSTRM)
        ],
        out_specs=pl.BlockSpec((1, 1, 2), lambda i: (i, 0, 0),
                               memory_space=pltpu.SMEM),
        out_shape=jax.ShapeDtypeStruct((_NBLK, 1, 2), jnp.float32),
        compiler_params=pltpu.CompilerParams(
            dimension_semantics=("parallel",),
        ),
    )(t3, *([output] * _NSTRM))
    res = pl.pallas_call(
        _combine_body,
        out_specs=pl.BlockSpec(memory_space=pltpu.SMEM),
        out_shape=jax.ShapeDtypeStruct((1, 1), jnp.float32),
    )(partials)
    return res[0, 0]
